# Initial kernel scaffold; baseline (speedup 1.0000x reference)
import jax
import jax.numpy as jnp
from jax.experimental import pallas as pl


def kernel(x, edge_index, edge_attr, batch_vec, W, emb, W_self, W_msg, W_edge, b):
    raise NotImplementedError("write your pallas kernel here")


# speedup vs baseline: 3.7757x; 3.7757x over previous
import functools

import jax
import jax.numpy as jnp
from jax import lax
from jax.experimental import pallas as pl
from jax.experimental.pallas import tpu as pltpu
from jax.experimental.pallas import tpu_sc as plsc

N_NODES = 10000
N_EDGES = 320000
D_EMB = 128
D_EDGE = 16
N_TYPES = 128

NC = 2
NS = 16
L = 16

EPT = 20480
E_PAD = NS * EPT
CH = 2048
NCHUNK = EPT // CH

TH = N_TYPES // NC
C_ROWS = 10048
C_FLAT = C_ROWS * TH
C_PER_TILE = C_FLAT // NS
INV_FLAT = 10032 * TH
E_ROWS = 10112
E_PER_TILE = E_ROWS // NS
ZBUF = 16384

ROW_BLK = 200
N_BLK = N_NODES // ROW_BLK


def _sc_body(src_hbm, dst_hbm, x_hbm, attr_hbm, cflat_hbm, eagg_hbm,
             x_v, src_v, dst_v, attr_v, fidx_v, didx_v, ones_v, zero_v,
             zeroe_v, c_sh, e_sh):
    cid = lax.axis_index("c")
    sid = lax.axis_index("s")

    def zb(i, carry):
        zero_v[pl.ds(i * L, L)] = jnp.zeros((L,), jnp.float32)
        return carry
    lax.fori_loop(0, ZBUF // L, zb, 0)

    def zbe(i, carry):
        zeroe_v[i, :] = jnp.zeros((D_EDGE,), jnp.float32)
        return carry
    lax.fori_loop(0, E_PER_TILE, zbe, 0)
    for g in range(128 // L):
        ones_v[pl.ds(g * L, L)] = jnp.ones((L,), jnp.float32)

    zbase = sid * C_PER_TILE
    for k in range(C_PER_TILE // ZBUF):
        pltpu.sync_copy(zero_v, c_sh.at[pl.ds(zbase + k * ZBUF, ZBUF)])
    rem = C_PER_TILE % ZBUF
    if rem:
        pltpu.sync_copy(zero_v.at[pl.ds(0, rem)],
                        c_sh.at[pl.ds(zbase + (C_PER_TILE // ZBUF) * ZBUF, rem)])
    pltpu.sync_copy(zeroe_v, e_sh.at[pl.ds(sid * E_PER_TILE, E_PER_TILE)])

    pltpu.sync_copy(x_hbm, x_v)

    plsc.subcore_barrier()

    def chunk(cc, carry):
        base = sid * EPT + cc * CH
        pltpu.sync_copy(src_hbm.at[pl.ds(base, CH)], src_v)
        pltpu.sync_copy(dst_hbm.at[pl.ds(base, CH)], dst_v)

        @pl.when(cid == 0)
        def _():
            pltpu.sync_copy(attr_hbm.at[pl.ds(base, CH)], attr_v)

        for g in range(CH // 128):
            def lane(j, carry2):
                i = g * 8 + j
                s16 = src_v[pl.ds(i * L, L)]
                d16 = dst_v[pl.ds(i * L, L)]
                t16 = plsc.load_gather(x_v, [s16])
                tloc = t16 - cid * TH
                mine = (t16 // TH) == cid
                fidx_v[g, pl.ds(j * L, L)] = jnp.where(
                    mine, d16 * TH + tloc, INV_FLAT)
                didx_v[g, pl.ds(j * L, L)] = d16
                return carry2
            lax.fori_loop(0, 8, lane, 0)
        for g in range(CH // 128):
            pltpu.sync_copy(ones_v, c_sh.at[fidx_v.at[g]], add=True)

        @pl.when(cid == 0)
        def _():
            for g in range(CH // 128):
                pltpu.sync_copy(attr_v.at[pl.ds(g * 128, 128)],
                                e_sh.at[didx_v.at[g]], add=True)
        return carry
    lax.fori_loop(0, NCHUNK, chunk, 0)

    plsc.subcore_barrier()

    pltpu.sync_copy(c_sh.at[pl.ds(sid * C_PER_TILE, C_PER_TILE)],
                    cflat_hbm.at[cid].at[pl.ds(sid * C_PER_TILE, C_PER_TILE)])

    @pl.when(cid == 0)
    def _():
        pltpu.sync_copy(e_sh.at[pl.ds(sid * E_PER_TILE, E_PER_TILE)],
                        eagg_hbm.at[pl.ds(sid * E_PER_TILE, E_PER_TILE)])


@functools.lru_cache(maxsize=1)
def _make_sc_build():
    return functools.partial(
        pl.kernel,
        out_type=(jax.ShapeDtypeStruct((NC, C_FLAT), jnp.float32),
                  jax.ShapeDtypeStruct((E_ROWS, D_EDGE), jnp.float32)),
        mesh=plsc.VectorSubcoreMesh(core_axis_name="c", subcore_axis_name="s",
                                    num_cores=NC, num_subcores=NS),
        scratch_types=[
            pltpu.VMEM((N_NODES,), jnp.int32),
            pltpu.VMEM((CH,), jnp.int32),
            pltpu.VMEM((CH,), jnp.int32),
            pltpu.VMEM((CH, D_EDGE), jnp.float32),
            pltpu.VMEM((CH // 128, 128), jnp.int32),
            pltpu.VMEM((CH // 128, 128), jnp.int32),
            pltpu.VMEM((128,), jnp.float32),
            pltpu.VMEM((ZBUF,), jnp.float32),
            pltpu.VMEM((E_PER_TILE, D_EDGE), jnp.float32),
            pltpu.VMEM_SHARED((C_FLAT,), jnp.float32),
            pltpu.VMEM_SHARED((E_ROWS, D_EDGE), jnp.float32),
        ],
        compiler_params=pltpu.CompilerParams(needs_layout_passes=False,
                                             use_tc_tiling_on_sc=False),
    )(_sc_body)


def _tc_body(x_ref, c_ref, e_ref, emb_ref, wself_ref, wmsg_ref, wedge_ref,
             b_ref, out_ref, hself_s, hmsg_s):
    @pl.when(pl.program_id(0) == 0)
    def _():
        hself_s[...] = jnp.dot(emb_ref[...], wself_ref[...],
                               preferred_element_type=jnp.float32)
        hmsg_s[...] = jnp.dot(emb_ref[...], wmsg_ref[...],
                              preferred_element_type=jnp.float32)

    xcol = x_ref[...]
    oh = (xcol == lax.broadcasted_iota(jnp.int32, (ROW_BLK, N_TYPES), 1)
          ).astype(jnp.float32)
    acc = jnp.dot(oh, hself_s[...], preferred_element_type=jnp.float32)
    acc = acc + jnp.dot(c_ref[0], hmsg_s[0:TH, :],
                        preferred_element_type=jnp.float32)
    acc = acc + jnp.dot(c_ref[1], hmsg_s[TH:N_TYPES, :],
                        preferred_element_type=jnp.float32)
    acc = acc + jnp.dot(e_ref[...], wedge_ref[...],
                        preferred_element_type=jnp.float32)
    out_ref[...] = jnp.maximum(acc + b_ref[...], 0.0)


def _tc_combine(xcol, cpart, eagg, emb, W_self, W_msg, W_edge, b2):
    return pl.pallas_call(
        _tc_body,
        grid=(N_BLK,),
        in_specs=[
            pl.BlockSpec((ROW_BLK, 1), lambda i: (i, 0)),
            pl.BlockSpec((NC, ROW_BLK, TH), lambda i: (0, i, 0)),
            pl.BlockSpec((ROW_BLK, D_EDGE), lambda i: (i, 0)),
            pl.BlockSpec((N_TYPES, D_EMB), lambda i: (0, 0)),
            pl.BlockSpec((D_EMB, D_EMB), lambda i: (0, 0)),
            pl.BlockSpec((D_EMB, D_EMB), lambda i: (0, 0)),
            pl.BlockSpec((D_EDGE, D_EMB), lambda i: (0, 0)),
            pl.BlockSpec((1, D_EMB), lambda i: (0, 0)),
        ],
        out_specs=pl.BlockSpec((ROW_BLK, D_EMB), lambda i: (i, 0)),
        out_shape=jax.ShapeDtypeStruct((N_NODES, D_EMB), jnp.float32),
        scratch_shapes=[pltpu.VMEM((N_TYPES, D_EMB), jnp.float32),
                        pltpu.VMEM((N_TYPES, D_EMB), jnp.float32)],
        compiler_params=pltpu.CompilerParams(
            dimension_semantics=("arbitrary",)),
    )(xcol, cpart, eagg, emb, W_self, W_msg, W_edge, b2)


def kernel(x, edge_index, edge_attr, batch_vec, W, emb, W_self, W_msg,
           W_edge, b):
    x = x.astype(jnp.int32)
    src = edge_index[0].astype(jnp.int32)
    dst = edge_index[1].astype(jnp.int32)
    pad = E_PAD - N_EDGES
    src_p = jnp.concatenate([src, jnp.zeros((pad,), jnp.int32)])
    dst_p = jnp.concatenate([dst, jnp.full((pad,), N_NODES, jnp.int32)])
    attr_p = jnp.concatenate(
        [edge_attr, jnp.zeros((pad, D_EDGE), jnp.float32)])

    cflat, eagg = _make_sc_build()(src_p, dst_p, x, attr_p)
    cpart = cflat.reshape(NC, C_ROWS, TH)

    return _tc_combine(x.reshape(N_NODES, 1), cpart, eagg, emb, W_self,
                       W_msg, W_edge, b.reshape(1, D_EMB))
